# Initial kernel scaffold; baseline (speedup 1.0000x reference)
import jax
import jax.numpy as jnp
from jax.experimental import pallas as pl


def kernel(inputs, target, mem_keys, W1, W2):
    raise NotImplementedError("write your pallas kernel here")


# speedup vs baseline: 1.9689x; 1.9689x over previous
import jax
import jax.numpy as jnp
from jax import lax
from jax.experimental import pallas as pl
from jax.experimental.pallas import tpu as pltpu

B = 64
D_IN = 8192
D_FEAT = 4096
D_OUT = 4096
K_MEM = 100
D_RETR = 7
BASE_LR = 2e-05

BF = 512
BO = 512
BI = 512

_F32 = jnp.float32


def _fwd1_body(x_ref, w1_ref, f_ref):
    f_ref[...] = jnp.dot(x_ref[...], w1_ref[...], preferred_element_type=_F32)


def _fwd2_body(f_ref, w2_ref, t_ref, o_ref, s_ref):
    j = pl.program_id(0)
    out = jnp.dot(f_ref[...], w2_ref[...], preferred_element_type=_F32)
    o_ref[...] = out
    t = t_ref[...]
    err = out - t

    @pl.when(j == 0)
    def _():
        s_ref[0, 0] = 0.0
        s_ref[0, 1] = 0.0

    s_ref[0, 0] += jnp.sum(err * err)
    s_ref[0, 1] += jnp.sum(t * t)


def _retrieve_body(f_ref, m_ref, lr_ref):
    feats = f_ref[...]
    M = m_ref[...]
    ksum = jnp.sum(M[:K_MEM - B, :], axis=0, keepdims=True) + jnp.sum(
        feats, axis=0, keepdims=True)
    k = ksum / K_MEM
    fn2 = jnp.sum(feats * feats, axis=1, keepdims=True)
    dots = jnp.sum(feats * k, axis=1, keepdims=True)
    kn = jnp.sqrt(jnp.sum(k * k))
    d = dots / (jnp.maximum(kn, 1e-8) * jnp.maximum(jnp.sqrt(fn2), 1e-8))

    d_row = jnp.transpose(d)
    less = (d_row < d).astype(_F32)
    ii = lax.broadcasted_iota(jnp.int32, (B, B), 0)
    jj = lax.broadcasted_iota(jnp.int32, (B, B), 1)
    eq_before = ((d_row == d) & (jj < ii)).astype(_F32)
    rank = jnp.sum(less + eq_before, axis=1, keepdims=True)
    w = (rank < D_RETR).astype(_F32)

    ssum = jnp.sum(w * M, axis=0, keepdims=True)
    sm = ssum / D_RETR
    smn = jnp.sqrt(jnp.sum(sm * sm))
    centers = sm / jnp.maximum(smn, 1e-12)
    rinv = 1.0 / jnp.maximum(jnp.sqrt(fn2), 1e-12)
    feats_n = jnp.sum(feats * rinv, axis=0, keepdims=True) / B
    fnn = jnp.sqrt(jnp.sum(feats_n * feats_n))
    cn = jnp.sqrt(jnp.sum(centers * centers))
    cos = jnp.sum(feats_n * centers) / (
        jnp.maximum(fnn, 1e-8) * jnp.maximum(cn, 1e-8))
    dist = 1.0 - cos
    lr_ref[0, 0] = BASE_LR * jnp.exp(-dist * 0.05)


def _bwd2_body(o_ref, t_ref, f_ref, w2_ref, c_ref, lr_ref, w2n_ref, gf_ref):
    c = c_ref[0, 0]
    lr = lr_ref[0, 0]
    g = c * (o_ref[...] - t_ref[...])
    fb = f_ref[...]
    w2b = w2_ref[...]
    gw2 = lax.dot_general(fb, g, (((0,), (0,)), ((), ())),
                          preferred_element_type=_F32)
    w2n_ref[...] = w2b - lr * gw2
    gf_ref[...] = lax.dot_general(g, w2b, (((1,), (1,)), ((), ())),
                                  preferred_element_type=_F32)


def _bwd1_body(x_ref, gf_ref, w1_ref, lr_ref, w1n_ref):
    lr = lr_ref[0, 0]
    xb = x_ref[...]
    gf = gf_ref[...]
    gw1 = lax.dot_general(xb, gf, (((0,), (0,)), ((), ())),
                          preferred_element_type=_F32)
    w1n_ref[...] = w1_ref[...] - lr * gw1


def kernel(inputs, target, mem_keys, W1, W2):
    feats = pl.pallas_call(
        _fwd1_body,
        grid=(D_FEAT // BF,),
        in_specs=[
            pl.BlockSpec((B, D_IN), lambda j: (0, 0)),
            pl.BlockSpec((D_IN, BF), lambda j: (0, j)),
        ],
        out_specs=pl.BlockSpec((B, BF), lambda j: (0, j)),
        out_shape=jax.ShapeDtypeStruct((B, D_FEAT), _F32),
    )(inputs, W1)

    outputs, sums = pl.pallas_call(
        _fwd2_body,
        grid=(D_OUT // BO,),
        in_specs=[
            pl.BlockSpec((B, D_FEAT), lambda j: (0, 0)),
            pl.BlockSpec((D_FEAT, BO), lambda j: (0, j)),
            pl.BlockSpec((B, BO), lambda j: (0, j)),
        ],
        out_specs=[
            pl.BlockSpec((B, BO), lambda j: (0, j)),
            pl.BlockSpec((1, 2), lambda j: (0, 0), memory_space=pltpu.SMEM),
        ],
        out_shape=[
            jax.ShapeDtypeStruct((B, D_OUT), _F32),
            jax.ShapeDtypeStruct((1, 2), _F32),
        ],
    )(feats, W2, target)

    n = jnp.float32(B * D_OUT)
    mse = sums[0, 0] / n
    mse_ref = jnp.maximum(sums[0, 1] / n, 1e-8)
    rmse = jnp.sqrt(mse)
    c = (0.5 / rmse + 1.0 / mse_ref) * 2.0 / n
    c = jnp.reshape(c, (1, 1))

    M = jnp.concatenate([mem_keys[B:], feats[:2 * B - K_MEM]], axis=0)
    lr = pl.pallas_call(
        _retrieve_body,
        in_specs=[
            pl.BlockSpec((B, D_FEAT), lambda: (0, 0)),
            pl.BlockSpec((B, D_FEAT), lambda: (0, 0)),
        ],
        out_specs=pl.BlockSpec(memory_space=pltpu.SMEM),
        out_shape=jax.ShapeDtypeStruct((1, 1), _F32),
    )(feats, M)

    W2_new, gradfeats = pl.pallas_call(
        _bwd2_body,
        grid=(D_FEAT // BF,),
        in_specs=[
            pl.BlockSpec((B, D_OUT), lambda j: (0, 0)),
            pl.BlockSpec((B, D_OUT), lambda j: (0, 0)),
            pl.BlockSpec((B, BF), lambda j: (0, j)),
            pl.BlockSpec((BF, D_OUT), lambda j: (j, 0)),
            pl.BlockSpec((1, 1), lambda j: (0, 0), memory_space=pltpu.SMEM),
            pl.BlockSpec((1, 1), lambda j: (0, 0), memory_space=pltpu.SMEM),
        ],
        out_specs=[
            pl.BlockSpec((BF, D_OUT), lambda j: (j, 0)),
            pl.BlockSpec((B, BF), lambda j: (0, j)),
        ],
        out_shape=[
            jax.ShapeDtypeStruct((D_FEAT, D_OUT), _F32),
            jax.ShapeDtypeStruct((B, D_FEAT), _F32),
        ],
    )(outputs, target, feats, W2, c, lr)

    W1_new = pl.pallas_call(
        _bwd1_body,
        grid=(D_IN // BI,),
        in_specs=[
            pl.BlockSpec((B, BI), lambda j: (0, j)),
            pl.BlockSpec((B, D_FEAT), lambda j: (0, 0)),
            pl.BlockSpec((BI, D_FEAT), lambda j: (j, 0)),
            pl.BlockSpec((1, 1), lambda j: (0, 0), memory_space=pltpu.SMEM),
        ],
        out_specs=pl.BlockSpec((BI, D_FEAT), lambda j: (j, 0)),
        out_shape=jax.ShapeDtypeStruct((D_IN, D_FEAT), _F32),
    )(inputs, gradfeats, W1, lr)

    adjusted_lr = jnp.reshape(lr, ())
    return outputs, adjusted_lr, W1_new, W2_new


# speedup vs baseline: 1.9706x; 1.0009x over previous
import jax
import jax.numpy as jnp
from jax import lax
from jax.experimental import pallas as pl
from jax.experimental.pallas import tpu as pltpu

B = 64
D_IN = 8192
D_FEAT = 4096
D_OUT = 4096
K_MEM = 100
D_RETR = 7
BASE_LR = 2e-05

BF = 512
BO = 512
BI = 512

_F32 = jnp.float32


def _fwd1_body(x_ref, w1_ref, f_ref):
    f_ref[...] = jnp.dot(x_ref[...], w1_ref[...], preferred_element_type=_F32)


def _fwd2_body(f_ref, w2_ref, t_ref, o_ref, s_ref):
    j = pl.program_id(0)
    out = jnp.dot(f_ref[...], w2_ref[...], preferred_element_type=_F32)
    o_ref[...] = out
    t = t_ref[...]
    err = out - t

    @pl.when(j == 0)
    def _():
        s_ref[0, 0] = 0.0
        s_ref[0, 1] = 0.0

    s_ref[0, 0] += jnp.sum(err * err)
    s_ref[0, 1] += jnp.sum(t * t)


def _retrieve_body(f_ref, m_ref, lr_ref):
    feats = f_ref[...]
    M = m_ref[...]
    ksum = jnp.sum(M[:K_MEM - B, :], axis=0, keepdims=True) + jnp.sum(
        feats, axis=0, keepdims=True)
    k = ksum / K_MEM
    fn2 = jnp.sum(feats * feats, axis=1, keepdims=True)
    dots = jnp.sum(feats * k, axis=1, keepdims=True)
    kn = jnp.sqrt(jnp.sum(k * k))
    d = dots / (jnp.maximum(kn, 1e-8) * jnp.maximum(jnp.sqrt(fn2), 1e-8))

    d_row = jnp.transpose(d)
    less = (d_row < d).astype(_F32)
    ii = lax.broadcasted_iota(jnp.int32, (B, B), 0)
    jj = lax.broadcasted_iota(jnp.int32, (B, B), 1)
    eq_before = ((d_row == d) & (jj < ii)).astype(_F32)
    rank = jnp.sum(less + eq_before, axis=1, keepdims=True)
    w = (rank < D_RETR).astype(_F32)

    ssum = jnp.sum(w * M, axis=0, keepdims=True)
    sm = ssum / D_RETR
    smn = jnp.sqrt(jnp.sum(sm * sm))
    centers = sm / jnp.maximum(smn, 1e-12)
    rinv = 1.0 / jnp.maximum(jnp.sqrt(fn2), 1e-12)
    feats_n = jnp.sum(feats * rinv, axis=0, keepdims=True) / B
    fnn = jnp.sqrt(jnp.sum(feats_n * feats_n))
    cn = jnp.sqrt(jnp.sum(centers * centers))
    cos = jnp.sum(feats_n * centers) / (
        jnp.maximum(fnn, 1e-8) * jnp.maximum(cn, 1e-8))
    dist = 1.0 - cos
    lr_ref[0, 0] = BASE_LR * jnp.exp(-dist * 0.05)


def _bwd2_body(o_ref, t_ref, f_ref, w2_ref, c_ref, lr_ref, w2n_ref, gf_ref):
    c = c_ref[0, 0]
    lr = lr_ref[0, 0]
    g = c * (o_ref[...] - t_ref[...])
    gh = g.astype(jnp.bfloat16)
    fb = f_ref[...].astype(jnp.bfloat16)
    w2b = w2_ref[...]
    gw2 = lax.dot_general(fb, gh, (((0,), (0,)), ((), ())),
                          preferred_element_type=_F32)
    w2n_ref[...] = w2b - lr * gw2
    gf_ref[...] = lax.dot_general(gh, w2b.astype(jnp.bfloat16),
                                  (((1,), (1,)), ((), ())),
                                  preferred_element_type=_F32)


def _bwd1_body(x_ref, gf_ref, w1_ref, lr_ref, w1n_ref):
    lr = lr_ref[0, 0]
    xb = x_ref[...].astype(jnp.bfloat16)
    gf = gf_ref[...].astype(jnp.bfloat16)
    gw1 = lax.dot_general(xb, gf, (((0,), (0,)), ((), ())),
                          preferred_element_type=_F32)
    w1n_ref[...] = w1_ref[...] - lr * gw1


def kernel(inputs, target, mem_keys, W1, W2):
    feats = pl.pallas_call(
        _fwd1_body,
        grid=(D_FEAT // BF,),
        in_specs=[
            pl.BlockSpec((B, D_IN), lambda j: (0, 0)),
            pl.BlockSpec((D_IN, BF), lambda j: (0, j)),
        ],
        out_specs=pl.BlockSpec((B, BF), lambda j: (0, j)),
        out_shape=jax.ShapeDtypeStruct((B, D_FEAT), _F32),
    )(inputs, W1)

    outputs, sums = pl.pallas_call(
        _fwd2_body,
        grid=(D_OUT // BO,),
        in_specs=[
            pl.BlockSpec((B, D_FEAT), lambda j: (0, 0)),
            pl.BlockSpec((D_FEAT, BO), lambda j: (0, j)),
            pl.BlockSpec((B, BO), lambda j: (0, j)),
        ],
        out_specs=[
            pl.BlockSpec((B, BO), lambda j: (0, j)),
            pl.BlockSpec((1, 2), lambda j: (0, 0), memory_space=pltpu.SMEM),
        ],
        out_shape=[
            jax.ShapeDtypeStruct((B, D_OUT), _F32),
            jax.ShapeDtypeStruct((1, 2), _F32),
        ],
    )(feats, W2, target)

    n = jnp.float32(B * D_OUT)
    mse = sums[0, 0] / n
    mse_ref = jnp.maximum(sums[0, 1] / n, 1e-8)
    rmse = jnp.sqrt(mse)
    c = (0.5 / rmse + 1.0 / mse_ref) * 2.0 / n
    c = jnp.reshape(c, (1, 1))

    M = jnp.concatenate([mem_keys[B:], feats[:2 * B - K_MEM]], axis=0)
    lr = pl.pallas_call(
        _retrieve_body,
        in_specs=[
            pl.BlockSpec((B, D_FEAT), lambda: (0, 0)),
            pl.BlockSpec((B, D_FEAT), lambda: (0, 0)),
        ],
        out_specs=pl.BlockSpec(memory_space=pltpu.SMEM),
        out_shape=jax.ShapeDtypeStruct((1, 1), _F32),
    )(feats, M)

    W2_new, gradfeats = pl.pallas_call(
        _bwd2_body,
        grid=(D_FEAT // BF,),
        in_specs=[
            pl.BlockSpec((B, D_OUT), lambda j: (0, 0)),
            pl.BlockSpec((B, D_OUT), lambda j: (0, 0)),
            pl.BlockSpec((B, BF), lambda j: (0, j)),
            pl.BlockSpec((BF, D_OUT), lambda j: (j, 0)),
            pl.BlockSpec((1, 1), lambda j: (0, 0), memory_space=pltpu.SMEM),
            pl.BlockSpec((1, 1), lambda j: (0, 0), memory_space=pltpu.SMEM),
        ],
        out_specs=[
            pl.BlockSpec((BF, D_OUT), lambda j: (j, 0)),
            pl.BlockSpec((B, BF), lambda j: (0, j)),
        ],
        out_shape=[
            jax.ShapeDtypeStruct((D_FEAT, D_OUT), _F32),
            jax.ShapeDtypeStruct((B, D_FEAT), _F32),
        ],
    )(outputs, target, feats, W2, c, lr)

    W1_new = pl.pallas_call(
        _bwd1_body,
        grid=(D_IN // BI,),
        in_specs=[
            pl.BlockSpec((B, BI), lambda j: (0, j)),
            pl.BlockSpec((B, D_FEAT), lambda j: (0, 0)),
            pl.BlockSpec((BI, D_FEAT), lambda j: (j, 0)),
            pl.BlockSpec((1, 1), lambda j: (0, 0), memory_space=pltpu.SMEM),
        ],
        out_specs=pl.BlockSpec((BI, D_FEAT), lambda j: (j, 0)),
        out_shape=jax.ShapeDtypeStruct((D_IN, D_FEAT), _F32),
    )(inputs, gradfeats, W1, lr)

    adjusted_lr = jnp.reshape(lr, ())
    return outputs, adjusted_lr, W1_new, W2_new


# speedup vs baseline: 1.9945x; 1.0121x over previous
import jax
import jax.numpy as jnp
from jax import lax
from jax.experimental import pallas as pl
from jax.experimental.pallas import tpu as pltpu

B = 64
D_IN = 8192
D_FEAT = 4096
D_OUT = 4096
K_MEM = 100
D_RETR = 7
BASE_LR = 2e-05

BF = 512
BO = 512
BI = 512

_F32 = jnp.float32


def _fwd1_body(x_ref, w1_ref, f_ref):
    f_ref[...] = jnp.dot(x_ref[...], w1_ref[...], preferred_element_type=_F32)


def _fwd2_body(f_ref, w2_ref, t_ref, o_ref, s_ref):
    j = pl.program_id(0)
    out = jnp.dot(f_ref[...], w2_ref[...], preferred_element_type=_F32)
    o_ref[...] = out
    t = t_ref[...]
    err = out - t

    @pl.when(j == 0)
    def _():
        s_ref[0, 0] = 0.0
        s_ref[0, 1] = 0.0

    s_ref[0, 0] += jnp.sum(err * err)
    s_ref[0, 1] += jnp.sum(t * t)


def _retrieve_body(f_ref, mk_ref, lr_ref):
    feats = f_ref[...]
    mk = mk_ref[...]
    M = jnp.concatenate([mk[B:], feats[:2 * B - K_MEM]], axis=0)
    ksum = jnp.sum(M[:K_MEM - B, :], axis=0, keepdims=True) + jnp.sum(
        feats, axis=0, keepdims=True)
    k = ksum / K_MEM
    fn2 = jnp.sum(feats * feats, axis=1, keepdims=True)
    dots = jnp.sum(feats * k, axis=1, keepdims=True)
    kn = jnp.sqrt(jnp.sum(k * k))
    d = dots / (jnp.maximum(kn, 1e-8) * jnp.maximum(jnp.sqrt(fn2), 1e-8))

    d_row = jnp.transpose(d)
    less = (d_row < d).astype(_F32)
    ii = lax.broadcasted_iota(jnp.int32, (B, B), 0)
    jj = lax.broadcasted_iota(jnp.int32, (B, B), 1)
    eq_before = ((d_row == d) & (jj < ii)).astype(_F32)
    rank = jnp.sum(less + eq_before, axis=1, keepdims=True)
    w = (rank < D_RETR).astype(_F32)

    ssum = jnp.sum(w * M, axis=0, keepdims=True)
    sm = ssum / D_RETR
    smn = jnp.sqrt(jnp.sum(sm * sm))
    centers = sm / jnp.maximum(smn, 1e-12)
    rinv = 1.0 / jnp.maximum(jnp.sqrt(fn2), 1e-12)
    feats_n = jnp.sum(feats * rinv, axis=0, keepdims=True) / B
    fnn = jnp.sqrt(jnp.sum(feats_n * feats_n))
    cn = jnp.sqrt(jnp.sum(centers * centers))
    cos = jnp.sum(feats_n * centers) / (
        jnp.maximum(fnn, 1e-8) * jnp.maximum(cn, 1e-8))
    dist = 1.0 - cos
    lr_ref[0, 0] = BASE_LR * jnp.exp(-dist * 0.05)


def _bwd2_body(o_ref, t_ref, f_ref, w2_ref, s_ref, lr_ref, w2n_ref, gf_ref):
    n = jnp.float32(B * D_OUT)
    mse = s_ref[0, 0] / n
    mse_ref = jnp.maximum(s_ref[0, 1] / n, 1e-8)
    c = (0.5 / jnp.sqrt(mse) + 1.0 / mse_ref) * 2.0 / n
    lr = lr_ref[0, 0]
    g = c * (o_ref[...] - t_ref[...])
    gh = g.astype(jnp.bfloat16)
    fb = f_ref[...].astype(jnp.bfloat16)
    w2b = w2_ref[...]
    gw2 = lax.dot_general(fb, gh, (((0,), (0,)), ((), ())),
                          preferred_element_type=_F32)
    w2n_ref[...] = w2b - lr * gw2
    gf_ref[...] = lax.dot_general(gh, w2b.astype(jnp.bfloat16),
                                  (((1,), (1,)), ((), ())),
                                  preferred_element_type=_F32)


def _bwd1_body(x_ref, gf_ref, w1_ref, lr_ref, w1n_ref):
    lr = lr_ref[0, 0]
    xb = x_ref[...].astype(jnp.bfloat16)
    gf = gf_ref[...].astype(jnp.bfloat16)
    gw1 = lax.dot_general(xb, gf, (((0,), (0,)), ((), ())),
                          preferred_element_type=_F32)
    w1n_ref[...] = w1_ref[...] - lr * gw1


def kernel(inputs, target, mem_keys, W1, W2):
    feats = pl.pallas_call(
        _fwd1_body,
        grid=(D_FEAT // BF,),
        in_specs=[
            pl.BlockSpec((B, D_IN), lambda j: (0, 0)),
            pl.BlockSpec((D_IN, BF), lambda j: (0, j)),
        ],
        out_specs=pl.BlockSpec((B, BF), lambda j: (0, j)),
        out_shape=jax.ShapeDtypeStruct((B, D_FEAT), _F32),
    )(inputs, W1)

    outputs, sums = pl.pallas_call(
        _fwd2_body,
        grid=(D_OUT // BO,),
        in_specs=[
            pl.BlockSpec((B, D_FEAT), lambda j: (0, 0)),
            pl.BlockSpec((D_FEAT, BO), lambda j: (0, j)),
            pl.BlockSpec((B, BO), lambda j: (0, j)),
        ],
        out_specs=[
            pl.BlockSpec((B, BO), lambda j: (0, j)),
            pl.BlockSpec((1, 2), lambda j: (0, 0), memory_space=pltpu.SMEM),
        ],
        out_shape=[
            jax.ShapeDtypeStruct((B, D_OUT), _F32),
            jax.ShapeDtypeStruct((1, 2), _F32),
        ],
    )(feats, W2, target)

    lr = pl.pallas_call(
        _retrieve_body,
        in_specs=[
            pl.BlockSpec((B, D_FEAT), lambda: (0, 0)),
            pl.BlockSpec((K_MEM, D_FEAT), lambda: (0, 0)),
        ],
        out_specs=pl.BlockSpec(memory_space=pltpu.SMEM),
        out_shape=jax.ShapeDtypeStruct((1, 1), _F32),
    )(feats, mem_keys)

    W2_new, gradfeats = pl.pallas_call(
        _bwd2_body,
        grid=(D_FEAT // BF,),
        in_specs=[
            pl.BlockSpec((B, D_OUT), lambda j: (0, 0)),
            pl.BlockSpec((B, D_OUT), lambda j: (0, 0)),
            pl.BlockSpec((B, BF), lambda j: (0, j)),
            pl.BlockSpec((BF, D_OUT), lambda j: (j, 0)),
            pl.BlockSpec((1, 2), lambda j: (0, 0), memory_space=pltpu.SMEM),
            pl.BlockSpec((1, 1), lambda j: (0, 0), memory_space=pltpu.SMEM),
        ],
        out_specs=[
            pl.BlockSpec((BF, D_OUT), lambda j: (j, 0)),
            pl.BlockSpec((B, BF), lambda j: (0, j)),
        ],
        out_shape=[
            jax.ShapeDtypeStruct((D_FEAT, D_OUT), _F32),
            jax.ShapeDtypeStruct((B, D_FEAT), _F32),
        ],
    )(outputs, target, feats, W2, sums, lr)

    W1_new = pl.pallas_call(
        _bwd1_body,
        grid=(D_IN // BI,),
        in_specs=[
            pl.BlockSpec((B, BI), lambda j: (0, j)),
            pl.BlockSpec((B, D_FEAT), lambda j: (0, 0)),
            pl.BlockSpec((BI, D_FEAT), lambda j: (j, 0)),
            pl.BlockSpec((1, 1), lambda j: (0, 0), memory_space=pltpu.SMEM),
        ],
        out_specs=pl.BlockSpec((BI, D_FEAT), lambda j: (j, 0)),
        out_shape=jax.ShapeDtypeStruct((D_IN, D_FEAT), _F32),
    )(inputs, gradfeats, W1, lr)

    adjusted_lr = jnp.reshape(lr, ())
    return outputs, adjusted_lr, W1_new, W2_new
